# Initial kernel scaffold; baseline (speedup 1.0000x reference)
import jax
import jax.numpy as jnp
from jax.experimental import pallas as pl


def kernel(codes, codebook, W, b, ln_gamma, ln_beta):
    raise NotImplementedError("write your pallas kernel here")


# speedup vs baseline: 3.3114x; 3.3114x over previous
import functools

import jax
import jax.numpy as jnp
from jax import lax
from jax.experimental import pallas as pl
from jax.experimental.pallas import tpu as pltpu
from jax.experimental.pallas import tpu_sc as plsc

B, N, NUM_CODES = 1024, 50, 8
VOCAB, RQVAE_DIM, LATENT_DIM = 1026, 256, 128
VPAD = 1032

NUM_TOKENS = B * N
NC, NS, L = 2, 16, 16
NW = NC * NS
TPW = NUM_TOKENS // NW
T_CHUNK = 16
N_CHUNKS = TPW // T_CHUNK
IDX_PER_CHUNK = T_CHUNK * NUM_CODES
C_CHUNKS = LATENT_DIM // L
LN_EPS = 1e-5


def _cw_body(cb_ref, w_ref, out_ref):
    out_ref[...] = jax.lax.dot(
        cb_ref[...], w_ref[...],
        precision=jax.lax.Precision.HIGHEST,
        preferred_element_type=jnp.float32,
    )


def _compute_cw(cb_pad, W):
    return pl.pallas_call(
        _cw_body,
        out_shape=jax.ShapeDtypeStruct((VPAD, LATENT_DIM), jnp.float32),
    )(cb_pad, W)


def _allsum16(v):
    iota = lax.iota(jnp.int32, L)
    for sh in (1, 2, 4, 8):
        v = v + jnp.take_along_axis(v, iota ^ sh, axis=0)
    return v


def _rsqrt16(x):
    i = lax.bitcast_convert_type(x, jnp.int32)
    y = lax.bitcast_convert_type(jnp.int32(0x5F3759DF) - (i >> 1), jnp.float32)
    for _ in range(3):
        y = y * (1.5 - 0.5 * x * y * y)
    return y


def _sc_body(cw_hbm, codes_hbm, params_hbm, out_hbm,
             idx_v, rows_v, out_v, params_v, sem):
    wid = lax.axis_index("s") * NC + lax.axis_index("c")
    pltpu.sync_copy(params_hbm, params_v)
    bias = [params_v[0, pl.ds(c * L, L)] for c in range(C_CHUNKS)]
    gamma = [params_v[1, pl.ds(c * L, L)] for c in range(C_CHUNKS)]
    beta = [params_v[2, pl.ds(c * L, L)] for c in range(C_CHUNKS)]
    tok0 = wid * TPW

    def chunk_body(ch, carry):
        base_tok = tok0 + ch * T_CHUNK
        pltpu.sync_copy(codes_hbm.at[pl.ds(base_tok * NUM_CODES, IDX_PER_CHUNK)],
                        idx_v)
        pltpu.async_copy(cw_hbm.at[idx_v], rows_v, sem).wait()
        for t in range(T_CHUNK):
            r0 = t * NUM_CODES
            xs = []
            for c in range(C_CHUNKS):
                s = c * L
                a0 = rows_v[r0 + 0, pl.ds(s, L)] + rows_v[r0 + 1, pl.ds(s, L)]
                a1 = rows_v[r0 + 2, pl.ds(s, L)] + rows_v[r0 + 3, pl.ds(s, L)]
                a2 = rows_v[r0 + 4, pl.ds(s, L)] + rows_v[r0 + 5, pl.ds(s, L)]
                a3 = rows_v[r0 + 6, pl.ds(s, L)] + rows_v[r0 + 7, pl.ds(s, L)]
                xs.append(((a0 + a1) + (a2 + a3)) + bias[c])
            tot = xs[0] + xs[1]
            for c in range(2, C_CHUNKS):
                tot = tot + xs[c]
            sq = xs[0] * xs[0] + xs[1] * xs[1]
            for c in range(2, C_CHUNKS):
                sq = sq + xs[c] * xs[c]
            mean_v = _allsum16(tot) * (1.0 / LATENT_DIM)
            ex2 = _allsum16(sq) * (1.0 / LATENT_DIM)
            var = ex2 - mean_v * mean_v
            rstd = _rsqrt16(var + LN_EPS)
            for c in range(C_CHUNKS):
                out_v[t, pl.ds(c * L, L)] = (
                    (xs[c] - mean_v) * rstd * gamma[c] + beta[c])
        pltpu.sync_copy(out_v, out_hbm.at[pl.ds(base_tok, T_CHUNK)])
        return carry

    lax.fori_loop(0, N_CHUNKS, chunk_body, 0)


def _sc_call(cw, codes_flat, params):
    mesh = plsc.VectorSubcoreMesh(
        core_axis_name="c", subcore_axis_name="s",
        num_cores=NC, num_subcores=NS)
    run = pl.kernel(
        _sc_body,
        out_type=jax.ShapeDtypeStruct((NUM_TOKENS, LATENT_DIM), jnp.float32),
        mesh=mesh,
        scratch_types=[
            pltpu.VMEM((IDX_PER_CHUNK,), jnp.int32),
            pltpu.VMEM((IDX_PER_CHUNK, LATENT_DIM), jnp.float32),
            pltpu.VMEM((T_CHUNK, LATENT_DIM), jnp.float32),
            pltpu.VMEM((3, LATENT_DIM), jnp.float32),
            pltpu.SemaphoreType.DMA,
        ],
    )
    return run(cw, codes_flat, params)


def kernel(codes, codebook, W, b, ln_gamma, ln_beta):
    cb_pad = jnp.concatenate(
        [codebook, jnp.zeros((VPAD - VOCAB, RQVAE_DIM), jnp.float32)], axis=0)
    cw = _compute_cw(cb_pad, W)
    codes_flat = codes.reshape(-1).astype(jnp.int32)
    params = jnp.stack([b, ln_gamma, ln_beta], axis=0)
    out = _sc_call(cw, codes_flat, params)
    return out.reshape(B, N, LATENT_DIM)
